# baseline (speedup 1.0000x reference)
import functools

import jax
import jax.numpy as jnp
from jax import lax
from jax.experimental import pallas as pl
from jax.experimental.pallas import tpu as pltpu
from jax.experimental.pallas import tpu_sc as plsc

_ROPE_BASE = 10000.0

_H0 = 104
_H1 = 96


def _sc_gather_rope(B, L, V, D):
    info = plsc.get_sparse_core_info()
    NC, NS, LN = info.num_cores, info.num_subcores, info.num_lanes
    NW = NC * NS
    seq_per_w = B // NW
    HN = [_H0, _H1]
    HOFF = [0, _H0]
    nj = D // LN
    CW = _H0 + _H1
    CS0 = CW
    P0 = 2 * CW

    mesh = plsc.VectorSubcoreMesh(core_axis_name="c", subcore_axis_name="s")

    @functools.partial(
        pl.kernel,
        mesh=mesh,
        out_type=jax.ShapeDtypeStruct((B, L, D), jnp.float32),
        scratch_types=[
            pltpu.VMEM((2 * 128,), jnp.int32),
            pltpu.VMEM((CW, 2 * D), jnp.float32),
            pltpu.VMEM((3 * CW, D), jnp.float32),
            pltpu.SemaphoreType.DMA((4,)),
        ],
    )
    def k(xp_hbm, emb_hbm, ctab_hbm, out_hbm, idx_v, rows_v, wc_v, sems):
        wid = lax.axis_index("s") * NC + lax.axis_index("c")
        seq0 = wid * seq_per_w
        pltpu.sync_copy(ctab_hbm, wc_v.at[pl.ds(CS0, 2 * CW)])

        def issue(t, h):
            sb = seq0 + t
            pltpu.sync_copy(xp_hbm.at[pl.ds((2 * sb + h) * 128, 128)],
                            idx_v.at[pl.ds(h * 128, 128)])
            pltpu.async_copy(
                emb_hbm.at[idx_v.at[pl.ds(h * 128, HN[h])]],
                rows_v.at[pl.ds(HOFF[h], HN[h])], sems.at[h])

        def drain_gathers(h):
            pltpu.make_async_copy(
                emb_hbm.at[idx_v.at[pl.ds(h * 128, HN[h])]],
                rows_v.at[pl.ds(HOFF[h], HN[h])], sems.at[h]).wait()

        def wait_write(h):
            pltpu.make_async_copy(
                wc_v.at[pl.ds(HOFF[h], HN[h])],
                out_hbm.at[seq0].at[pl.ds(HOFF[h], HN[h])],
                sems.at[2 + h]).wait()

        def compute(h):
            def row_body(r, carry):
                pos = HOFF[h] + r
                rr = HOFF[h] + r
                rb = [rows_v[rr, pl.ds(j * LN, LN)] for j in range(nj)]
                cs = [wc_v[CS0 + pos, pl.ds(j * LN, LN)] for j in range(nj)]
                for j in range(nj):
                    js = (j + nj // 2) % nj
                    cj = cs[j % 2]
                    sj = cs[2 + (j % 2)]
                    pj = wc_v[P0 + pos, pl.ds(j * LN, LN)]
                    if j < nj // 2:
                        wc_v[rr, pl.ds(j * LN, LN)] = (
                            rb[j] * cj - rb[js] * sj + pj)
                    else:
                        wc_v[rr, pl.ds(j * LN, LN)] = (
                            rb[js] * sj + rb[j] * cj + pj)
                return carry
            lax.fori_loop(0, HN[h], row_body, 0)

        def write(t, h):
            pltpu.async_copy(
                wc_v.at[pl.ds(HOFF[h], HN[h])],
                out_hbm.at[seq0 + t].at[pl.ds(HOFF[h], HN[h])],
                sems.at[2 + h])

        issue(0, 0)

        def seq_body(t2, carry):
            drain_gathers(0)

            @pl.when(t2 > 0)
            def _():
                wait_write(1)

            issue(t2, 1)
            compute(0)
            write(t2, 0)

            drain_gathers(1)

            @pl.when(t2 < seq_per_w - 1)
            def _():
                wait_write(0)
                issue(t2 + 1, 0)

            compute(1)
            write(t2, 1)
            return carry

        lax.fori_loop(0, seq_per_w, seq_body, 0)
        wait_write(0)
        wait_write(1)

    return k


def kernel(x, emb_table, pos_table):
    B, L = x.shape
    V, D = emb_table.shape
    half = D // 2
    embdup = jnp.concatenate([emb_table, emb_table], axis=1)
    xi = x.astype(jnp.int32)
    xh = jnp.zeros((B, 2, 128), jnp.int32)
    xh = xh.at[:, 0, :_H0].set(xi[:, :_H0])
    xh = xh.at[:, 1, :_H1].set(xi[:, _H0:])
    xp = xh.reshape(-1)
    freqs = 1.0 / (_ROPE_BASE ** (jnp.arange(half, dtype=jnp.float32) / D))
    ang = jnp.arange(L, dtype=jnp.float32)[:, None] * freqs[None, :]
    cs = jnp.concatenate([jnp.cos(ang), jnp.sin(ang)], axis=-1)
    ctab = jnp.concatenate([cs, pos_table.astype(jnp.float32)], axis=0)
    return _sc_gather_rope(B, L, V, D)(xp, embdup, ctab)


# speedup vs baseline: 1.0122x; 1.0122x over previous
import functools

import jax
import jax.numpy as jnp
from jax import lax
from jax.experimental import pallas as pl
from jax.experimental.pallas import tpu as pltpu
from jax.experimental.pallas import tpu_sc as plsc

_ROPE_BASE = 10000.0


def _sc_gather_rope(BL, V, D, L):
    info = plsc.get_sparse_core_info()
    NC, NS, LN = info.num_cores, info.num_subcores, info.num_lanes
    NW = NC * NS
    assert BL % NW == 0
    per_w = BL // NW
    C = L
    assert per_w % (2 * C) == 0
    n_chunks = per_w // C
    half_n = n_chunks // 2
    G = 40
    assert C % G == 0
    n_g = C // G
    nj = D // LN

    mesh = plsc.VectorSubcoreMesh(core_axis_name="c", subcore_axis_name="s")

    @functools.partial(
        pl.kernel,
        mesh=mesh,
        compiler_params=pltpu.CompilerParams(use_tc_tiling_on_sc=False),
        out_type=jax.ShapeDtypeStruct((BL, D), jnp.float32),
        scratch_types=[
            pltpu.VMEM((C,), jnp.int32),
            pltpu.VMEM((C,), jnp.int32),
            pltpu.VMEM((C, D), jnp.float32),
            pltpu.VMEM((C, D), jnp.float32),
            pltpu.VMEM((L, D), jnp.float32),
            pltpu.VMEM((L, D), jnp.float32),
            pltpu.VMEM((L, D), jnp.float32),
            pltpu.SemaphoreType.DMA,
            pltpu.SemaphoreType.DMA,
            pltpu.SemaphoreType.DMA,
            pltpu.SemaphoreType.DMA,
        ],
    )
    def k(idx_hbm, emb_hbm, c1_hbm, c2_hbm, p_hbm, out_hbm,
          idx0, idx1, rows0, rows1, c1_v, c2_v, p_v,
          gsem0, gsem1, wsem0, wsem1):
        wid = lax.axis_index("s") * NC + lax.axis_index("c")
        base_w = wid * per_w
        pltpu.sync_copy(c1_hbm, c1_v)
        pltpu.sync_copy(c2_hbm, c2_v)
        pltpu.sync_copy(p_hbm, p_v)

        def issue(t, idx_v, rows_v, gsem):
            base = base_w + t * C
            pltpu.sync_copy(idx_hbm.at[pl.ds(base, C)], idx_v)
            for g in range(n_g):
                pltpu.async_copy(
                    emb_hbm.at[idx_v.at[pl.ds(g * G, G)]],
                    rows_v.at[pl.ds(g * G, G)], gsem)

        def drain_gathers(idx_v, rows_v, gsem):
            pltpu.make_async_copy(emb_hbm.at[idx_v], rows_v, gsem).wait()

        def wait_write(rows_v, wsem):
            pltpu.make_async_copy(
                rows_v, out_hbm.at[pl.ds(base_w, C)], wsem).wait()

        def compute(rows_v):
            def row_body(r, carry):
                rb = [rows_v[r, pl.ds(j * LN, LN)] for j in range(nj)]
                for j in range(nj):
                    js = (j + nj // 2) % nj
                    rows_v[r, pl.ds(j * LN, LN)] = (
                        rb[j] * c1_v[r, pl.ds(j * LN, LN)]
                        + rb[js] * c2_v[r, pl.ds(j * LN, LN)]
                        + p_v[r, pl.ds(j * LN, LN)])
                return carry
            lax.fori_loop(0, C, row_body, 0)

        def write(t, rows_v, wsem):
            pltpu.async_copy(rows_v, out_hbm.at[pl.ds(base_w + t * C, C)],
                             wsem)

        issue(0, idx0, rows0, gsem0)

        def pair_body(t2, carry):
            te = 2 * t2

            drain_gathers(idx0, rows0, gsem0)

            @pl.when(t2 > 0)
            def _():
                wait_write(rows1, wsem1)

            issue(te + 1, idx1, rows1, gsem1)
            compute(rows0)
            write(te, rows0, wsem0)

            drain_gathers(idx1, rows1, gsem1)

            @pl.when(t2 < half_n - 1)
            def _():
                wait_write(rows0, wsem0)
                issue(te + 2, idx0, rows0, gsem0)

            compute(rows1)
            write(te + 1, rows1, wsem1)
            return carry

        lax.fori_loop(0, half_n, pair_body, 0)
        wait_write(rows0, wsem0)
        wait_write(rows1, wsem1)

    return k


def kernel(x, emb_table, pos_table):
    B, L = x.shape
    V, D = emb_table.shape
    half = D // 2
    idx = x.reshape(B * L).astype(jnp.int32)
    freqs = 1.0 / (_ROPE_BASE ** (jnp.arange(half, dtype=jnp.float32) / D))
    ang = jnp.arange(L, dtype=jnp.float32)[:, None] * freqs[None, :]
    c = jnp.cos(ang)
    s = jnp.sin(ang)
    c1 = jnp.concatenate([c, c], axis=-1)
    c2 = jnp.concatenate([-s, s], axis=-1)
    out = _sc_gather_rope(B * L, V, D, L)(
        idx, emb_table, c1, c2, pos_table.astype(jnp.float32))
    return out.reshape(B, L, D)
